# Initial kernel scaffold; baseline (speedup 1.0000x reference)
import jax
import jax.numpy as jnp
from jax.experimental import pallas as pl


def kernel(x, W_g, W_noise, W1, b1, W2, b2, k):
    raise NotImplementedError("write your pallas kernel here")


# speedup vs baseline: 1.3101x; 1.3101x over previous
import functools

import jax
import jax.numpy as jnp
from jax.experimental import pallas as pl
from jax.experimental.pallas import tpu as pltpu

_B, _S, _D, _E = 2, 2048, 768, 8
_N = _B * _S


def _gating_body(x_ref, wg_ref, wn_ref, nc_ref, g_ref):
    x = x_ref[...]
    prelim = jnp.dot(x, wg_ref[...], preferred_element_type=jnp.float32)
    noise = jax.nn.softplus(
        jnp.dot(x, wn_ref[...], preferred_element_type=jnp.float32))
    h = prelim + nc_ref[...] + noise
    m1 = jnp.max(h, axis=-1, keepdims=True)
    is_max = h == m1
    nmax = jnp.sum(is_max.astype(jnp.float32), axis=-1, keepdims=True)
    m2 = jnp.max(jnp.where(is_max, -jnp.inf, h), axis=-1, keepdims=True)
    thresh = jnp.where(nmax >= 2.0, m1, m2)
    hm = jnp.where(h >= thresh, h, -jnp.inf)
    z = jnp.exp(hm - m1)
    g_ref[...] = z / jnp.sum(z, axis=-1, keepdims=True)


def _expert_body(g_ref, x_ref, w1_ref, b1_ref, w2_ref, b2_ref, o_ref):
    e = pl.program_id(1)
    x = x_ref[...]
    h = jnp.dot(x, w1_ref[0], preferred_element_type=jnp.float32)
    h = jnp.maximum(h + b1_ref[0], 0.0)
    out = jnp.dot(h, w2_ref[0], preferred_element_type=jnp.float32)
    out = out + b2_ref[0]
    lane = jax.lax.broadcasted_iota(jnp.int32, g_ref.shape, 1)
    gate = jnp.sum(jnp.where(lane == e, g_ref[...], 0.0), axis=1,
                   keepdims=True)
    contrib = out * gate

    @pl.when(e == 0)
    def _():
        o_ref[...] = contrib

    @pl.when(e > 0)
    def _():
        o_ref[...] += contrib


@functools.partial(jax.jit, static_argnames=("interpret",))
def _moe(x, W_g, W_noise, W1, b1, W2, b2, interpret=False):
    xf = x.reshape(_N, _D)
    nconst = jax.random.normal(jax.random.key(42), (_B, _S, _E),
                               dtype=jnp.float32).reshape(_N, _E)

    TG = 2048
    gates = pl.pallas_call(
        _gating_body,
        grid=(_N // TG,),
        in_specs=[
            pl.BlockSpec((TG, _D), lambda t: (t, 0)),
            pl.BlockSpec((_D, _E), lambda t: (0, 0)),
            pl.BlockSpec((_D, _E), lambda t: (0, 0)),
            pl.BlockSpec((TG, _E), lambda t: (t, 0)),
        ],
        out_specs=pl.BlockSpec((TG, _E), lambda t: (t, 0)),
        out_shape=jax.ShapeDtypeStruct((_N, _E), jnp.float32),
        interpret=interpret,
    )(xf, W_g, W_noise, nconst)

    TM = 1024
    b1r = b1.reshape(_E, 1, _D)
    b2r = b2.reshape(_E, 1, _D)
    y = pl.pallas_call(
        _expert_body,
        grid=(_N // TM, _E),
        in_specs=[
            pl.BlockSpec((TM, _E), lambda t, e: (t, 0)),
            pl.BlockSpec((TM, _D), lambda t, e: (t, 0)),
            pl.BlockSpec((1, _D, _D), lambda t, e: (e, 0, 0)),
            pl.BlockSpec((1, 1, _D), lambda t, e: (e, 0, 0)),
            pl.BlockSpec((1, _D, _D), lambda t, e: (e, 0, 0)),
            pl.BlockSpec((1, 1, _D), lambda t, e: (e, 0, 0)),
        ],
        out_specs=pl.BlockSpec((TM, _D), lambda t, e: (t, 0)),
        out_shape=jax.ShapeDtypeStruct((_N, _D), jnp.float32),
        interpret=interpret,
    )(gates, xf, W1, b1r, W2, b2r)
    return y.reshape(_B, _S, _D)


def kernel(x, W_g, W_noise, W1, b1, W2, b2, k):
    return _moe(x, W_g, W_noise, W1, b1, W2, b2)
